# Initial kernel scaffold; baseline (speedup 1.0000x reference)
import jax
import jax.numpy as jnp
from jax.experimental import pallas as pl


def kernel(inputs, token_table, pos_table):
    raise NotImplementedError("write your pallas kernel here")


# speedup vs baseline: 1.4893x; 1.4893x over previous
import functools

import jax
import jax.numpy as jnp
from jax import lax
from jax.experimental import pallas as pl
from jax.experimental.pallas import tpu as pltpu
from jax.experimental.pallas import tpu_sc as plsc

_BATCH = 4096
_L = 200
_D = 32
_NB = 4
_G = 100
_HALVES = _L // _G


@functools.cache
def _build_sc_call():
    info = plsc.get_sparse_core_info()
    nc, ns = info.num_cores, info.num_subcores
    nw = nc * ns
    seq_w = _BATCH // nw
    chunks_w = seq_w * _HALVES
    passes = seq_w // _NB

    mesh = plsc.VectorSubcoreMesh(core_axis_name="c", subcore_axis_name="s")

    scratch = (
        [pltpu.VMEM((chunks_w, _G), jnp.int32),
         pltpu.VMEM((_L, _D), jnp.float32)]
        + [pltpu.VMEM((_L, _D), jnp.float32) for _ in range(2 * _NB)]
        + [pltpu.SemaphoreType.DMA for _ in range(2 * _NB)]
    )

    @functools.partial(
        pl.kernel,
        mesh=mesh,
        out_type=jax.ShapeDtypeStruct((_BATCH, _L, _D), jnp.float32),
        scratch_types=scratch,
        compiler_params=pltpu.CompilerParams(use_tc_tiling_on_sc=False),
    )
    def emb_kernel(idx_hbm, tok_hbm, pos_hbm, out_hbm, *sc):
        idx_v, pos_v = sc[0], sc[1]
        gbufs = sc[2:2 + _NB]
        obufs = sc[2 + _NB:2 + 2 * _NB]
        gsems = sc[2 + 2 * _NB:2 + 3 * _NB]
        osems = sc[2 + 3 * _NB:2 + 4 * _NB]

        wid = lax.axis_index("s") * nc + lax.axis_index("c")
        seq0 = wid * seq_w

        pltpu.sync_copy(idx_hbm.at[wid], idx_v)
        pltpu.sync_copy(pos_hbm, pos_v)

        def start_gathers(b, s):
            for h in range(_HALVES):
                pltpu.async_copy(
                    tok_hbm.at[idx_v.at[_HALVES * s + h]],
                    gbufs[b].at[pl.ds(h * _G, _G), :],
                    gsems[b],
                )

        def wait_gathers(b, s):
            for h in range(_HALVES):
                pltpu.make_async_copy(
                    tok_hbm.at[idx_v.at[_HALVES * s + h]],
                    gbufs[b].at[pl.ds(h * _G, _G), :],
                    gsems[b],
                ).wait()

        for b in range(_NB):
            start_gathers(b, b)

        def one_pass(g, carry):
            for b in range(_NB):
                s = g * _NB + b
                wait_gathers(b, s)

                @pl.when(g > 0)
                def _wait_prev_out(b=b, s=s):
                    pltpu.make_async_copy(
                        obufs[b], out_hbm.at[seq0 + s - _NB], osems[b]
                    ).wait()

                def add_body(t, c, b=b):
                    for u in range(8):
                        tt = t * 8 + u
                        for hh in range(_D // 16):
                            sl = pl.ds(hh * 16, 16)
                            obufs[b][tt, sl] = gbufs[b][tt, sl] + pos_v[tt, sl]
                    return c

                lax.fori_loop(0, _L // 8, add_body, 0)

                pltpu.async_copy(obufs[b], out_hbm.at[seq0 + s], osems[b])

                @pl.when(g < passes - 1)
                def _prefetch_next(b=b, s=s):
                    start_gathers(b, s + _NB)
            return carry

        lax.fori_loop(0, passes, one_pass, 0)

        for b in range(_NB):
            s = (passes - 1) * _NB + b
            pltpu.make_async_copy(
                obufs[b], out_hbm.at[seq0 + s], osems[b]
            ).wait()

    return emb_kernel, nw, chunks_w


def kernel(inputs, token_table, pos_table):
    emb, nw, chunks_w = _build_sc_call()
    idx = inputs.astype(jnp.int32).reshape(nw, chunks_w, _G)
    return emb(idx, token_table, pos_table)
